# baseline (speedup 1.0000x reference)
import functools

import jax
import jax.numpy as jnp
from jax.experimental import pallas as pl
from jax.experimental.pallas import tpu as pltpu

O = 12
V = 3
P = O * (O - 1) * (O - 2)
P0, U, BD = 16, 32, 16
R, C = 6, 64
RC = R * C
D = P0 + V * U + V * (V - 1) * BD

_HI = jax.lax.Precision.DEFAULT


def _dnf_kernel(nul_ref, una_ref, bin_ref, w_ref, bias_ref,
                orn_ref, oru_ref, orb_ref,
                conj_ref, outn_ref, outu_ref, outb_ref):
    w = w_ref[...]
    bias = bias_ref[...]

    base = jax.lax.dot_general(nul_ref[0], w[0:P0],
                               (((1,), (0,)), ((), ())),
                               precision=_HI) + bias
    una = una_ref[0]
    wu = jnp.concatenate(
        [w[P0 + U * v: P0 + U * (v + 1)] for v in range(V)], axis=1)
    u_cat = jax.lax.dot_general(una, wu, (((1,), (0,)), ((), ())),
                                precision=_HI)
    u_v = [u_cat[:, RC * v: RC * (v + 1)] for v in range(V)]
    binb = bin_ref[0]
    boff = P0 + V * U
    wb = jnp.concatenate(
        [w[boff + BD * s: boff + BD * (s + 1)] for s in range(6)], axis=1)
    bp_cat = jax.lax.dot_general(binb, wb, (((1,), (0,)), ((), ())),
                                 precision=_HI)
    bp = [bp_cat[:, RC * s: RC * (s + 1)] for s in range(6)]

    def pair_row(fwd, rev, i):
        parts = []
        for j in range(O):
            if j == i:
                continue
            jj = j - (j > i)
            ii = i - (i > j)
            parts.append(fwd[i * (O - 1) + jj: i * (O - 1) + jj + 1]
                         + rev[j * (O - 1) + ii: j * (O - 1) + ii + 1])
        return jnp.concatenate(parts, axis=0)

    A_rows = [pair_row(bp[0], bp[2], i) for i in range(O)]
    B_rows = [pair_row(bp[1], bp[4], i) for i in range(O)]
    C_rows = [pair_row(bp[3], bp[5], i) for i in range(O)]

    null_maxes, u_rows, b_rows = [], [], []
    for o0 in range(O):
        others0 = [o for o in range(O) if o != o0]
        g_base = base + u_v[0][o0:o0 + 1]
        h = jnp.concatenate(
            [u_v[2][o2:o2 + 1] for o2 in others0], axis=0) + B_rows[o0]
        blocks = []
        for r1, o1 in enumerate(others0):
            def drop_row(arr, pos, n):
                parts = ([arr[:pos]] if pos > 0 else []) + \
                        ([arr[pos + 1:]] if pos < n - 1 else [])
                return parts[0] if len(parts) == 1 else \
                    jnp.concatenate(parts, axis=0)

            pos_o0 = o0 - (o0 > o1)
            c_sel = drop_row(C_rows[o1], pos_o0, O - 1)
            h_sel = drop_row(h, r1, O - 1)
            row = (g_base + u_v[1][o1:o1 + 1]
                   + A_rows[o0][r1:r1 + 1]) + h_sel + c_sel
            blocks.append(row)
        conj_o0 = jnp.tanh(jnp.concatenate(blocks, axis=0))
        conj_ref[0, o0 * 110:(o0 + 1) * 110, :, :] = conj_o0.reshape(110, R, C)
        null_maxes.append(jnp.max(conj_o0[:, 0:2 * C], axis=0, keepdims=True))
        u_rows.append(jnp.max(conj_o0[:, 2 * C:4 * C], axis=0, keepdims=True))
        for r1 in range(O - 1):
            b_rows.append(jnp.max(
                conj_o0[r1 * 10:(r1 + 1) * 10, 4 * C:6 * C],
                axis=0, keepdims=True))

    null_max = functools.reduce(jnp.maximum, null_maxes)
    u_rules = jnp.concatenate(u_rows, axis=0)
    b_rules = jnp.concatenate(b_rows, axis=0)

    def disjoin(rules, or_ref):
        sig = jax.nn.sigmoid(or_ref[...])
        prod = rules * sig
        s0 = jnp.sum(prod[:, 0:C], axis=1, keepdims=True)
        s1 = jnp.sum(prod[:, C:2 * C], axis=1, keepdims=True)
        return jnp.tanh(jnp.concatenate([s0, s1], axis=1))

    outn_ref[0] = disjoin(null_max, orn_ref)
    outu_ref[0] = disjoin(u_rules, oru_ref)
    outb_ref[0] = disjoin(b_rules, orb_ref)


def kernel(nullary, unary, binary, and_kernel, and_bias,
           or_nullary, or_unary, or_binary):
    B = nullary.shape[0]
    w = and_kernel.transpose(2, 0, 1).reshape(D, RC)
    bias = and_bias.reshape(1, RC)
    bin2 = binary.reshape(B, O * (O - 1), BD)
    orn = or_nullary.reshape(1, 2 * C)
    oru = or_unary.reshape(1, 2 * C)
    orb = or_binary.reshape(1, 2 * C)
    nul3 = nullary.reshape(B, 1, P0)

    conj, outn, outu, outb = pl.pallas_call(
        _dnf_kernel,
        grid=(B,),
        in_specs=[
            pl.BlockSpec((1, 1, P0), lambda b: (b, 0, 0)),
            pl.BlockSpec((1, O, U), lambda b: (b, 0, 0)),
            pl.BlockSpec((1, O * (O - 1), BD), lambda b: (b, 0, 0)),
            pl.BlockSpec((D, RC), lambda b: (0, 0)),
            pl.BlockSpec((1, RC), lambda b: (0, 0)),
            pl.BlockSpec((1, 2 * C), lambda b: (0, 0)),
            pl.BlockSpec((1, 2 * C), lambda b: (0, 0)),
            pl.BlockSpec((1, 2 * C), lambda b: (0, 0)),
        ],
        out_specs=[
            pl.BlockSpec((1, P, R, C), lambda b: (b, 0, 0, 0)),
            pl.BlockSpec((1, 1, 2), lambda b: (b, 0, 0)),
            pl.BlockSpec((1, O, 2), lambda b: (b, 0, 0)),
            pl.BlockSpec((1, O * (O - 1), 2), lambda b: (b, 0, 0)),
        ],
        out_shape=[
            jax.ShapeDtypeStruct((B, P, R, C), jnp.float32),
            jax.ShapeDtypeStruct((B, 1, 2), jnp.float32),
            jax.ShapeDtypeStruct((B, O, 2), jnp.float32),
            jax.ShapeDtypeStruct((B, O * (O - 1), 2), jnp.float32),
        ],
        compiler_params=pltpu.CompilerParams(
            dimension_semantics=("parallel",)),
    )(nul3, unary, bin2, w, bias, orn, oru, orb)

    conjuncts = conj
    out_binary = outb.reshape(B, O, O - 1, 2)
    return (outn.reshape(B, 2), outu, out_binary, conjuncts)


# speedup vs baseline: 1.5885x; 1.5885x over previous
import functools

import jax
import jax.numpy as jnp
from jax.experimental import pallas as pl
from jax.experimental.pallas import tpu as pltpu

O = 12
V = 3
P = O * (O - 1) * (O - 2)
P0, U, BD = 16, 32, 16
R, C = 6, 64
RC = R * C
D = P0 + V * U + V * (V - 1) * BD
BB = 2

_DOT = jax.lax.Precision.DEFAULT


def _dnf_body(nul, una, binb, wu_aug, wb,
              orn_ref, oru_ref, orb_ref,
              conj_ref, outn_ref, outu_ref, outb_ref, bi):
    ones = jnp.ones((O, 1), jnp.float32)
    una_aug = jnp.concatenate(
        [una, jnp.broadcast_to(nul, (O, P0)), ones], axis=1)
    u_cat = jax.lax.dot_general(una_aug, wu_aug, (((1,), (0,)), ((), ())),
                                precision=_DOT)
    u_v = [u_cat[:, RC * v: RC * (v + 1)] for v in range(V)]
    bp_cat = jax.lax.dot_general(binb, wb, (((1,), (0,)), ((), ())),
                                 precision=_DOT)
    bp = [bp_cat[:, RC * s: RC * (s + 1)] for s in range(6)]

    def pair_row(fwd, rev, i):
        parts = []
        for j in range(O):
            if j == i:
                continue
            jj = j - (j > i)
            ii = i - (i > j)
            parts.append(fwd[i * (O - 1) + jj: i * (O - 1) + jj + 1]
                         + rev[j * (O - 1) + ii: j * (O - 1) + ii + 1])
        return jnp.concatenate(parts, axis=0)

    A_rows = [pair_row(bp[0], bp[2], i) for i in range(O)]
    B_rows = [pair_row(bp[1], bp[4], i) for i in range(O)]
    C_rows = [pair_row(bp[3], bp[5], i) for i in range(O)]

    def drop_row(arr, pos, n):
        parts = ([arr[:pos]] if pos > 0 else []) + \
                ([arr[pos + 1:]] if pos < n - 1 else [])
        return parts[0] if len(parts) == 1 else \
            jnp.concatenate(parts, axis=0)

    null_maxes, u_rows, b_rows = [], [], []
    for o0 in range(O):
        others0 = [o for o in range(O) if o != o0]
        g_base = u_v[0][o0:o0 + 1]
        ga = g_base + drop_row(u_v[1], o0, O) + A_rows[o0]
        h = drop_row(u_v[2], o0, O) + B_rows[o0]
        blocks = []
        for r1, o1 in enumerate(others0):
            pos_o0 = o0 - (o0 > o1)
            c_sel = drop_row(C_rows[o1], pos_o0, O - 1)
            h_sel = drop_row(h, r1, O - 1)
            blocks.append(ga[r1:r1 + 1] + h_sel + c_sel)
        conj_o0 = jnp.tanh(jnp.concatenate(blocks, axis=0))
        conj_ref[bi, o0 * 110:(o0 + 1) * 110, :] = conj_o0
        null_maxes.append(jnp.max(conj_o0[:, 0:2 * C], axis=0, keepdims=True))
        u_rows.append(jnp.max(conj_o0[:, 2 * C:4 * C], axis=0, keepdims=True))
        for r1 in range(O - 1):
            b_rows.append(jnp.max(
                conj_o0[r1 * 10:(r1 + 1) * 10, 4 * C:6 * C],
                axis=0, keepdims=True))

    null_max = functools.reduce(jnp.maximum, null_maxes)
    u_rules = jnp.concatenate(u_rows, axis=0)
    b_rules = jnp.concatenate(b_rows, axis=0)

    def disjoin(rules, or_ref):
        sig = jax.nn.sigmoid(or_ref[...])
        prod = rules * sig
        s0 = jnp.sum(prod[:, 0:C], axis=1, keepdims=True)
        s1 = jnp.sum(prod[:, C:2 * C], axis=1, keepdims=True)
        return jnp.tanh(jnp.concatenate([s0, s1], axis=1))

    outn_ref[bi] = disjoin(null_max, orn_ref)
    outu_ref[bi] = disjoin(u_rules, oru_ref)
    outb_ref[bi] = disjoin(b_rules, orb_ref)


def _dnf_kernel(nul_ref, una_ref, bin_ref, wu_ref, wb_ref,
                orn_ref, oru_ref, orb_ref,
                conj_ref, outn_ref, outu_ref, outb_ref):
    wu_aug = wu_ref[...]
    wb = wb_ref[...]
    for bi in range(BB):
        _dnf_body(nul_ref[bi], una_ref[bi], bin_ref[bi], wu_aug, wb,
                  orn_ref, oru_ref, orb_ref,
                  conj_ref, outn_ref, outu_ref, outb_ref, bi)


def kernel(nullary, unary, binary, and_kernel, and_bias,
           or_nullary, or_unary, or_binary):
    B = nullary.shape[0]
    w = and_kernel.transpose(2, 0, 1).reshape(D, RC)
    wu = jnp.concatenate(
        [w[P0 + U * v: P0 + U * (v + 1)] for v in range(V)], axis=1)
    zpad = jnp.zeros((P0 + 1, 2 * RC), jnp.float32)
    extra = jnp.concatenate(
        [jnp.concatenate([w[0:P0], and_bias.reshape(1, RC)], axis=0), zpad],
        axis=1)
    wu_aug = jnp.concatenate([wu, extra], axis=0)
    boff = P0 + V * U
    wb = jnp.concatenate(
        [w[boff + BD * s: boff + BD * (s + 1)] for s in range(6)], axis=1)
    bin2 = binary.reshape(B, O * (O - 1), BD)
    orn = or_nullary.reshape(1, 2 * C)
    oru = or_unary.reshape(1, 2 * C)
    orb = or_binary.reshape(1, 2 * C)
    nul3 = nullary.reshape(B, 1, P0)

    conj, outn, outu, outb = pl.pallas_call(
        _dnf_kernel,
        grid=(B // BB,),
        in_specs=[
            pl.BlockSpec((BB, 1, P0), lambda b: (b, 0, 0)),
            pl.BlockSpec((BB, O, U), lambda b: (b, 0, 0)),
            pl.BlockSpec((BB, O * (O - 1), BD), lambda b: (b, 0, 0)),
            pl.BlockSpec((U + P0 + 1, V * RC), lambda b: (0, 0)),
            pl.BlockSpec((BD, 6 * RC), lambda b: (0, 0)),
            pl.BlockSpec((1, 2 * C), lambda b: (0, 0)),
            pl.BlockSpec((1, 2 * C), lambda b: (0, 0)),
            pl.BlockSpec((1, 2 * C), lambda b: (0, 0)),
        ],
        out_specs=[
            pl.BlockSpec((BB, P, RC), lambda b: (b, 0, 0)),
            pl.BlockSpec((BB, 1, 2), lambda b: (b, 0, 0)),
            pl.BlockSpec((BB, O, 2), lambda b: (b, 0, 0)),
            pl.BlockSpec((BB, O * (O - 1), 2), lambda b: (b, 0, 0)),
        ],
        out_shape=[
            jax.ShapeDtypeStruct((B, P, RC), jnp.float32),
            jax.ShapeDtypeStruct((B, 1, 2), jnp.float32),
            jax.ShapeDtypeStruct((B, O, 2), jnp.float32),
            jax.ShapeDtypeStruct((B, O * (O - 1), 2), jnp.float32),
        ],
        compiler_params=pltpu.CompilerParams(
            dimension_semantics=("parallel",)),
    )(nul3, unary, bin2, wu_aug, wb, orn, oru, orb)

    conjuncts = conj.reshape(B, P, R, C)
    out_binary = outb.reshape(B, O, O - 1, 2)
    return (outn.reshape(B, 2), outu, out_binary, conjuncts)


# speedup vs baseline: 1.8394x; 1.1579x over previous
import functools

import jax
import jax.numpy as jnp
import numpy as np
from jax.experimental import pallas as pl
from jax.experimental.pallas import tpu as pltpu

O = 12
V = 3
P = O * (O - 1) * (O - 2)
P0, U, BD = 16, 32, 16
R, C = 6, 64
RC = R * C
D = P0 + V * U + V * (V - 1) * BD
BB = 4

_DOT = jax.lax.Precision.DEFAULT


def _pmat_np():
    m = np.zeros((132, 132), np.float32)
    for i in range(O):
        for j in range(O):
            if i == j:
                continue
            dst = i * (O - 1) + j - (j > i)
            src = j * (O - 1) + i - (i > j)
            m[dst, src] = 1.0
    return m


def _sel_np():
    m = np.zeros((264, 36), np.float32)
    for o0 in range(O):
        others0 = [o for o in range(O) if o != o0]
        for r1, o1 in enumerate(others0):
            m[o0 * 11 + r1, o0] = 1.0
            m[o0 * 11 + r1, 12 + o1] = 1.0
            m[132 + o0 * 11 + r1, 24 + o1] = 1.0
    return m


def _dnf_body(nul, una, binb, wu_aug, wb, pmat, sel,
              orn_ref, oru_ref, orb_ref,
              conj_ref, outn_ref, outu_ref, outb_ref, bi):
    ones = jnp.ones((O, 1), jnp.float32)
    una_aug = jnp.concatenate(
        [una, jnp.broadcast_to(nul, (O, P0)), ones], axis=1)
    u_cat = jax.lax.dot_general(una_aug, wu_aug, (((1,), (0,)), ((), ())),
                                precision=_DOT)
    x = jnp.concatenate(
        [u_cat[:, 0:RC], u_cat[:, RC:2 * RC], u_cat[:, 2 * RC:3 * RC]],
        axis=0)
    bp_cat = jax.lax.dot_general(binb, wb, (((1,), (0,)), ((), ())),
                                 precision=_DOT)
    bp_perm = jax.lax.dot_general(pmat, bp_cat[:, 3 * RC:6 * RC],
                                  (((1,), (0,)), ((), ())),
                                  precision=_DOT)
    abc = bp_cat[:, 0:3 * RC] + bp_perm
    y = jax.lax.dot_general(sel, x, (((1,), (0,)), ((), ())),
                            precision=_DOT)
    ga_all = y[0:132] + abc[:, 0:RC]
    h_all = y[132:264] + abc[:, RC:2 * RC]
    c_flat = abc[:, 2 * RC:3 * RC]

    def drop_row(arr, pos, n):
        parts = ([arr[:pos]] if pos > 0 else []) + \
                ([arr[pos + 1:]] if pos < n - 1 else [])
        return parts[0] if len(parts) == 1 else \
            jnp.concatenate(parts, axis=0)

    null_maxes, u_rows, b_rows = [], [], []
    for o0 in range(O):
        others0 = [o for o in range(O) if o != o0]
        ga = ga_all[o0 * 11:(o0 + 1) * 11]
        h = h_all[o0 * 11:(o0 + 1) * 11]
        blocks = []
        for r1, o1 in enumerate(others0):
            pos_o0 = o0 - (o0 > o1)
            c_sel = drop_row(c_flat[o1 * 11:(o1 + 1) * 11], pos_o0, O - 1)
            h_sel = drop_row(h, r1, O - 1)
            blocks.append(ga[r1:r1 + 1] + h_sel + c_sel)
        conj_o0 = jnp.tanh(jnp.concatenate(blocks, axis=0))
        conj_ref[bi, o0 * 110:(o0 + 1) * 110, :] = conj_o0.astype(jnp.bfloat16)
        null_maxes.append(jnp.max(conj_o0[:, 0:2 * C], axis=0, keepdims=True))
        u_rows.append(jnp.max(conj_o0[:, 2 * C:4 * C], axis=0, keepdims=True))
        for r1 in range(O - 1):
            b_rows.append(jnp.max(
                conj_o0[r1 * 10:(r1 + 1) * 10, 4 * C:6 * C],
                axis=0, keepdims=True))

    null_max = functools.reduce(jnp.maximum, null_maxes)
    u_rules = jnp.concatenate(u_rows, axis=0)
    b_rules = jnp.concatenate(b_rows, axis=0)

    def disjoin(rules, or_ref):
        sig = jax.nn.sigmoid(or_ref[...])
        prod = rules * sig
        s0 = jnp.sum(prod[:, 0:C], axis=1, keepdims=True)
        s1 = jnp.sum(prod[:, C:2 * C], axis=1, keepdims=True)
        return jnp.tanh(jnp.concatenate([s0, s1], axis=1))

    outn_ref[bi] = disjoin(null_max, orn_ref)
    outu_ref[bi] = disjoin(u_rules, oru_ref)
    outb_ref[bi] = disjoin(b_rules, orb_ref)


def _dnf_kernel(nul_ref, una_ref, bin_ref, wu_ref, wb_ref, pmat_ref, sel_ref,
                orn_ref, oru_ref, orb_ref,
                conj_ref, outn_ref, outu_ref, outb_ref):
    wu_aug = wu_ref[...]
    wb = wb_ref[...]
    pmat = pmat_ref[...]
    sel = sel_ref[...]
    for bi in range(BB):
        _dnf_body(nul_ref[bi], una_ref[bi], bin_ref[bi], wu_aug, wb,
                  pmat, sel, orn_ref, oru_ref, orb_ref,
                  conj_ref, outn_ref, outu_ref, outb_ref, bi)


def kernel(nullary, unary, binary, and_kernel, and_bias,
           or_nullary, or_unary, or_binary):
    B = nullary.shape[0]
    w = and_kernel.transpose(2, 0, 1).reshape(D, RC)
    wu = jnp.concatenate(
        [w[P0 + U * v: P0 + U * (v + 1)] for v in range(V)], axis=1)
    zpad = jnp.zeros((P0 + 1, 2 * RC), jnp.float32)
    extra = jnp.concatenate(
        [jnp.concatenate([w[0:P0], and_bias.reshape(1, RC)], axis=0), zpad],
        axis=1)
    wu_aug = jnp.concatenate([wu, extra], axis=0)
    boff = P0 + V * U
    wb = jnp.concatenate(
        [w[boff + BD * s: boff + BD * (s + 1)] for s in (0, 1, 3, 2, 4, 5)],
        axis=1)
    pmat = jnp.asarray(_pmat_np())
    sel = jnp.asarray(_sel_np())
    bin2 = binary.reshape(B, O * (O - 1), BD)
    orn = or_nullary.reshape(1, 2 * C)
    oru = or_unary.reshape(1, 2 * C)
    orb = or_binary.reshape(1, 2 * C)
    nul3 = nullary.reshape(B, 1, P0)

    conj, outn, outu, outb = pl.pallas_call(
        _dnf_kernel,
        grid=(B // BB,),
        in_specs=[
            pl.BlockSpec((BB, 1, P0), lambda b: (b, 0, 0)),
            pl.BlockSpec((BB, O, U), lambda b: (b, 0, 0)),
            pl.BlockSpec((BB, O * (O - 1), BD), lambda b: (b, 0, 0)),
            pl.BlockSpec((U + P0 + 1, V * RC), lambda b: (0, 0)),
            pl.BlockSpec((BD, 6 * RC), lambda b: (0, 0)),
            pl.BlockSpec((132, 132), lambda b: (0, 0)),
            pl.BlockSpec((264, 36), lambda b: (0, 0)),
            pl.BlockSpec((1, 2 * C), lambda b: (0, 0)),
            pl.BlockSpec((1, 2 * C), lambda b: (0, 0)),
            pl.BlockSpec((1, 2 * C), lambda b: (0, 0)),
        ],
        out_specs=[
            pl.BlockSpec((BB, P, RC), lambda b: (b, 0, 0)),
            pl.BlockSpec((BB, 1, 2), lambda b: (b, 0, 0)),
            pl.BlockSpec((BB, O, 2), lambda b: (b, 0, 0)),
            pl.BlockSpec((BB, O * (O - 1), 2), lambda b: (b, 0, 0)),
        ],
        out_shape=[
            jax.ShapeDtypeStruct((B, P, RC), jnp.bfloat16),
            jax.ShapeDtypeStruct((B, 1, 2), jnp.float32),
            jax.ShapeDtypeStruct((B, O, 2), jnp.float32),
            jax.ShapeDtypeStruct((B, O * (O - 1), 2), jnp.float32),
        ],
        compiler_params=pltpu.CompilerParams(
            dimension_semantics=("parallel",)),
    )(nul3, unary, bin2, wu_aug, wb, pmat, sel, orn, oru, orb)

    conjuncts = conj.reshape(B, P, R, C).astype(jnp.float32)
    out_binary = outb.reshape(B, O, O - 1, 2)
    return (outn.reshape(B, 2), outu, out_binary, conjuncts)


# speedup vs baseline: 1.8530x; 1.0074x over previous
import functools

import jax
import jax.numpy as jnp
import numpy as np
from jax.experimental import pallas as pl
from jax.experimental.pallas import tpu as pltpu

O = 12
V = 3
P = O * (O - 1) * (O - 2)
P0, U, BD = 16, 32, 16
R, C = 6, 64
RC = R * C
D = P0 + V * U + V * (V - 1) * BD
BB = 8

_DOT = jax.lax.Precision.DEFAULT


def _pmat_np():
    m = np.zeros((132, 132), np.float32)
    for i in range(O):
        for j in range(O):
            if i == j:
                continue
            dst = i * (O - 1) + j - (j > i)
            src = j * (O - 1) + i - (i > j)
            m[dst, src] = 1.0
    return m


def _sel_np():
    m = np.zeros((264, 36), np.float32)
    for o0 in range(O):
        others0 = [o for o in range(O) if o != o0]
        for r1, o1 in enumerate(others0):
            m[o0 * 11 + r1, o0] = 1.0
            m[o0 * 11 + r1, 12 + o1] = 1.0
            m[132 + o0 * 11 + r1, 24 + o1] = 1.0
    return m


def _dnf_body(nul, una, binb, wu_aug, wb, pmat, sel,
              orn_ref, oru_ref, orb_ref,
              conj_ref, outn_ref, outu_ref, outb_ref, bi):
    ones = jnp.ones((O, 1), jnp.float32)
    una_aug = jnp.concatenate(
        [una, jnp.broadcast_to(nul, (O, P0)), ones], axis=1)
    u_cat = jax.lax.dot_general(una_aug, wu_aug, (((1,), (0,)), ((), ())),
                                precision=_DOT)
    x = jnp.concatenate(
        [u_cat[:, 0:RC], u_cat[:, RC:2 * RC], u_cat[:, 2 * RC:3 * RC]],
        axis=0)
    bp_cat = jax.lax.dot_general(binb, wb, (((1,), (0,)), ((), ())),
                                 precision=_DOT)
    bp_perm = jax.lax.dot_general(pmat, bp_cat[:, 3 * RC:6 * RC],
                                  (((1,), (0,)), ((), ())),
                                  precision=_DOT)
    abc = bp_cat[:, 0:3 * RC] + bp_perm
    y = jax.lax.dot_general(sel, x, (((1,), (0,)), ((), ())),
                            precision=_DOT)
    ga_all = y[0:132] + abc[:, 0:RC]
    h_all = y[132:264] + abc[:, RC:2 * RC]
    c_flat = abc[:, 2 * RC:3 * RC]

    def drop_row(arr, pos, n):
        parts = ([arr[:pos]] if pos > 0 else []) + \
                ([arr[pos + 1:]] if pos < n - 1 else [])
        return parts[0] if len(parts) == 1 else \
            jnp.concatenate(parts, axis=0)

    null_maxes, u_rows, b_rows = [], [], []
    for o0 in range(O):
        others0 = [o for o in range(O) if o != o0]
        ga = ga_all[o0 * 11:(o0 + 1) * 11]
        h = h_all[o0 * 11:(o0 + 1) * 11]
        blocks = []
        for r1, o1 in enumerate(others0):
            pos_o0 = o0 - (o0 > o1)
            c_sel = drop_row(c_flat[o1 * 11:(o1 + 1) * 11], pos_o0, O - 1)
            h_sel = drop_row(h, r1, O - 1)
            blocks.append(ga[r1:r1 + 1] + h_sel + c_sel)
        conj_o0 = jnp.tanh(jnp.concatenate(blocks, axis=0))
        conj_ref[bi, o0 * 110:(o0 + 1) * 110, :] = conj_o0.astype(jnp.bfloat16)
        null_maxes.append(jnp.max(conj_o0[:, 0:2 * C], axis=0, keepdims=True))
        u_rows.append(jnp.max(conj_o0[:, 2 * C:4 * C], axis=0, keepdims=True))
        for r1 in range(O - 1):
            b_rows.append(jnp.max(
                conj_o0[r1 * 10:(r1 + 1) * 10, 4 * C:6 * C],
                axis=0, keepdims=True))

    null_max = functools.reduce(jnp.maximum, null_maxes)
    u_rules = jnp.concatenate(u_rows, axis=0)
    b_rules = jnp.concatenate(b_rows, axis=0)

    def disjoin(rules, or_ref):
        sig = jax.nn.sigmoid(or_ref[...])
        prod = rules * sig
        s0 = jnp.sum(prod[:, 0:C], axis=1, keepdims=True)
        s1 = jnp.sum(prod[:, C:2 * C], axis=1, keepdims=True)
        return jnp.tanh(jnp.concatenate([s0, s1], axis=1))

    outn_ref[bi] = disjoin(null_max, orn_ref)
    outu_ref[bi] = disjoin(u_rules, oru_ref)
    outb_ref[bi] = disjoin(b_rules, orb_ref)


def _dnf_kernel(nul_ref, una_ref, bin_ref, wu_ref, wb_ref, pmat_ref, sel_ref,
                orn_ref, oru_ref, orb_ref,
                conj_ref, outn_ref, outu_ref, outb_ref):
    wu_aug = wu_ref[...]
    wb = wb_ref[...]
    pmat = pmat_ref[...]
    sel = sel_ref[...]
    for bi in range(BB):
        _dnf_body(nul_ref[bi], una_ref[bi], bin_ref[bi], wu_aug, wb,
                  pmat, sel, orn_ref, oru_ref, orb_ref,
                  conj_ref, outn_ref, outu_ref, outb_ref, bi)


def kernel(nullary, unary, binary, and_kernel, and_bias,
           or_nullary, or_unary, or_binary):
    B = nullary.shape[0]
    w = and_kernel.transpose(2, 0, 1).reshape(D, RC)
    wu = jnp.concatenate(
        [w[P0 + U * v: P0 + U * (v + 1)] for v in range(V)], axis=1)
    zpad = jnp.zeros((P0 + 1, 2 * RC), jnp.float32)
    extra = jnp.concatenate(
        [jnp.concatenate([w[0:P0], and_bias.reshape(1, RC)], axis=0), zpad],
        axis=1)
    wu_aug = jnp.concatenate([wu, extra], axis=0)
    boff = P0 + V * U
    wb = jnp.concatenate(
        [w[boff + BD * s: boff + BD * (s + 1)] for s in (0, 1, 3, 2, 4, 5)],
        axis=1)
    pmat = jnp.asarray(_pmat_np())
    sel = jnp.asarray(_sel_np())
    bin2 = binary.reshape(B, O * (O - 1), BD)
    orn = or_nullary.reshape(1, 2 * C)
    oru = or_unary.reshape(1, 2 * C)
    orb = or_binary.reshape(1, 2 * C)
    nul3 = nullary.reshape(B, 1, P0)

    conj, outn, outu, outb = pl.pallas_call(
        _dnf_kernel,
        grid=(B // BB,),
        in_specs=[
            pl.BlockSpec((BB, 1, P0), lambda b: (b, 0, 0)),
            pl.BlockSpec((BB, O, U), lambda b: (b, 0, 0)),
            pl.BlockSpec((BB, O * (O - 1), BD), lambda b: (b, 0, 0)),
            pl.BlockSpec((U + P0 + 1, V * RC), lambda b: (0, 0)),
            pl.BlockSpec((BD, 6 * RC), lambda b: (0, 0)),
            pl.BlockSpec((132, 132), lambda b: (0, 0)),
            pl.BlockSpec((264, 36), lambda b: (0, 0)),
            pl.BlockSpec((1, 2 * C), lambda b: (0, 0)),
            pl.BlockSpec((1, 2 * C), lambda b: (0, 0)),
            pl.BlockSpec((1, 2 * C), lambda b: (0, 0)),
        ],
        out_specs=[
            pl.BlockSpec((BB, P, RC), lambda b: (b, 0, 0)),
            pl.BlockSpec((BB, 1, 2), lambda b: (b, 0, 0)),
            pl.BlockSpec((BB, O, 2), lambda b: (b, 0, 0)),
            pl.BlockSpec((BB, O * (O - 1), 2), lambda b: (b, 0, 0)),
        ],
        out_shape=[
            jax.ShapeDtypeStruct((B, P, RC), jnp.bfloat16),
            jax.ShapeDtypeStruct((B, 1, 2), jnp.float32),
            jax.ShapeDtypeStruct((B, O, 2), jnp.float32),
            jax.ShapeDtypeStruct((B, O * (O - 1), 2), jnp.float32),
        ],
        compiler_params=pltpu.CompilerParams(
            dimension_semantics=("parallel",)),
    )(nul3, unary, bin2, wu_aug, wb, pmat, sel, orn, oru, orb)

    conjuncts = conj.reshape(B, P, R, C).astype(jnp.float32)
    out_binary = outb.reshape(B, O, O - 1, 2)
    return (outn.reshape(B, 2), outu, out_binary, conjuncts)
